# baseline (speedup 1.0000x reference)
import functools

import jax
import jax.numpy as jnp
import numpy as np
from jax import lax
from jax.experimental import pallas as pl
from jax.experimental.pallas import tpu as pltpu

N, D, P, H = 8, 32, 4096, 1
HD = D // H
VOXEL_BASE = 0.05
VOXEL_RANGE = 0.1
BIN_SIZE = 10

BQ = 512
NQB = P // BQ
_INV_SQRT_D = np.float32(0.176776692)


def _attn_body(pts_ref, wv_ref, wk_ref, wq_ref, wo_ref, bo_ref,
               out_ref, voxel_ref, bins_ref,
               qt_ref, kt_ref, vt_ref, cm_ref, gmm_ref):
    n = pl.program_id(0)
    qb = pl.program_id(1)
    pts = pts_ref[0]

    @pl.when(qb == 0)
    def _project():
        ptsb = pts.astype(jnp.bfloat16)
        qt_ref[...] = jax.lax.dot_general(
            wq_ref[...].astype(jnp.bfloat16), ptsb, (((1,), (0,)), ((), ())),
            preferred_element_type=jnp.float32).astype(jnp.bfloat16)
        kt_ref[...] = jax.lax.dot_general(
            wk_ref[...].astype(jnp.bfloat16), ptsb, (((1,), (0,)), ((), ())),
            preferred_element_type=jnp.float32).astype(jnp.bfloat16)
        vt_ref[...] = jax.lax.dot_general(
            wv_ref[...], pts, (((1,), (0,)), ((), ())),
            preferred_element_type=jnp.float32,
            precision=jax.lax.Precision.HIGHEST)

    q_blk = qt_ref[:, pl.ds(qb * BQ, BQ)]
    e = jax.lax.dot_general(
        q_blk, kt_ref[...], (((0,), (0,)), ((), ())),
        preferred_element_type=jnp.float32) * _INV_SQRT_D
    m = jnp.max(e, axis=1, keepdims=True)
    p = jnp.exp(e - m)
    l = jnp.sum(p, axis=1, keepdims=True)
    att = p / l

    cm_blk = jnp.max(att, axis=0, keepdims=True)

    @pl.when(qb == 0)
    def _cm_init():
        cm_ref[...] = cm_blk

    @pl.when(qb > 0)
    def _cm_acc():
        cm_ref[...] = jnp.maximum(cm_ref[...], cm_blk)

    ov = jax.lax.dot_general(
        att, vt_ref[...], (((1,), (1,)), ((), ())),
        preferred_element_type=jnp.float32,
        precision=jax.lax.Precision.HIGHEST)
    out_ref[0] = jax.lax.dot_general(
        ov, wo_ref[...], (((1,), (1,)), ((), ())),
        preferred_element_type=jnp.float32,
        precision=jax.lax.Precision.HIGHEST) + bo_ref[...]

    @pl.when(qb == NQB - 1)
    def _voxel():
        cm = cm_ref[...]
        mn = jnp.min(cm)
        mx = jnp.max(cm)
        norm = (cm - mn) / (mx - mn)
        voxel = VOXEL_BASE + (1.0 - norm) * VOXEL_RANGE
        voxel_ref[0] = voxel
        vmn = jnp.min(voxel)
        vmx = jnp.max(voxel)

        @pl.when(n == 0)
        def _g_init():
            gmm_ref[0] = vmn
            gmm_ref[1] = vmx

        @pl.when(n > 0)
        def _g_acc():
            gmm_ref[0] = jnp.minimum(gmm_ref[0], vmn)
            gmm_ref[1] = jnp.maximum(gmm_ref[1], vmx)

        @pl.when(n == N - 1)
        def _bins():
            i_f = lax.broadcasted_iota(jnp.int32, (1, 128), 1).astype(jnp.float32)
            step = i_f / np.float32(BIN_SIZE)
            bins_ref[...] = gmm_ref[0] * (1.0 - step) + gmm_ref[1] * step


def _attention_call(batched_pts, Wv, Wk, Wq, Wo, bo2, interpret=False):
    return pl.pallas_call(
        _attn_body,
        grid=(N, NQB),
        in_specs=[
            pl.BlockSpec((1, D, P), lambda n, q: (n, 0, 0)),
            pl.BlockSpec((D, D), lambda n, q: (0, 0)),
            pl.BlockSpec((D, D), lambda n, q: (0, 0)),
            pl.BlockSpec((D, D), lambda n, q: (0, 0)),
            pl.BlockSpec((D, D), lambda n, q: (0, 0)),
            pl.BlockSpec((1, D), lambda n, q: (0, 0)),
        ],
        out_specs=[
            pl.BlockSpec((1, BQ, D), lambda n, q: (n, q, 0)),
            pl.BlockSpec((1, 1, P), lambda n, q: (n, 0, 0)),
            pl.BlockSpec((1, 128), lambda n, q: (0, 0)),
        ],
        out_shape=[
            jax.ShapeDtypeStruct((N, P, D), jnp.float32),
            jax.ShapeDtypeStruct((N, 1, P), jnp.float32),
            jax.ShapeDtypeStruct((1, 128), jnp.float32),
        ],
        scratch_shapes=[
            pltpu.VMEM((D, P), jnp.bfloat16),
            pltpu.VMEM((D, P), jnp.bfloat16),
            pltpu.VMEM((D, P), jnp.float32),
            pltpu.VMEM((1, P), jnp.float32),
            pltpu.SMEM((2,), jnp.float32),
        ],
        interpret=interpret,
    )(batched_pts, Wv, Wk, Wq, Wo, bo2)


def kernel(batched_pts, Wv, Wk, Wq, Wo, bo):
    out, voxel3, bins128 = _attention_call(
        batched_pts, Wv, Wk, Wq, Wo, bo.reshape(1, D))
    voxel_sizes = voxel3.reshape(N, P)

    bins = bins128[0, :BIN_SIZE + 1]
    flat_vs = voxel_sizes.reshape(-1)
    bin_index = jnp.searchsorted(bins, flat_vs, side='right') - 1
    bin_index = jnp.where((bin_index < 0) | (bin_index >= BIN_SIZE),
                          BIN_SIZE - 1, bin_index)
    counts = jnp.bincount(bin_index, length=BIN_SIZE)
    xyz_flat = jnp.transpose(batched_pts, (0, 2, 1)).reshape(-1, D)
    bin_sums = jax.ops.segment_sum(xyz_flat, bin_index, num_segments=BIN_SIZE)
    return out, voxel_sizes, counts, bin_sums


# speedup vs baseline: 1.5676x; 1.5676x over previous
import functools

import jax
import jax.numpy as jnp
import numpy as np
from jax import lax
from jax.experimental import pallas as pl
from jax.experimental.pallas import tpu as pltpu

N, D, P, H = 8, 32, 4096, 1
HD = D // H
VOXEL_BASE = 0.05
VOXEL_RANGE = 0.1
BIN_SIZE = 10

BQ = 512
NQB = P // BQ
_INV_SQRT_D = np.float32(0.176776692)


def _attn_body(pts_ref, wv_ref, wk_ref, wq_ref, wo_ref, bo_ref,
               out_ref, voxel_ref, bins_ref,
               qt_ref, kt_ref, vt_ref, cm_ref, gmm_ref):
    n = pl.program_id(0)
    qb = pl.program_id(1)
    pts = pts_ref[0]

    @pl.when(qb == 0)
    def _project():
        ptsb = pts.astype(jnp.bfloat16)
        qt_ref[...] = jax.lax.dot_general(
            wq_ref[...].astype(jnp.bfloat16), ptsb, (((1,), (0,)), ((), ())),
            preferred_element_type=jnp.float32).astype(jnp.bfloat16)
        kt_ref[...] = jax.lax.dot_general(
            wk_ref[...].astype(jnp.bfloat16), ptsb, (((1,), (0,)), ((), ())),
            preferred_element_type=jnp.float32).astype(jnp.bfloat16)
        vt_ref[...] = jax.lax.dot_general(
            wv_ref[...], pts, (((1,), (0,)), ((), ())),
            preferred_element_type=jnp.float32,
            precision=jax.lax.Precision.HIGHEST)

    q_blk = qt_ref[:, pl.ds(qb * BQ, BQ)]
    e = jax.lax.dot_general(
        q_blk, kt_ref[...], (((0,), (0,)), ((), ())),
        preferred_element_type=jnp.float32) * _INV_SQRT_D
    m = jnp.max(e, axis=1, keepdims=True)
    p = jnp.exp(e - m)
    l = jnp.sum(p, axis=1, keepdims=True)
    att = p * (1.0 / l)

    cm_blk = jnp.max(att, axis=0, keepdims=True)

    @pl.when(qb == 0)
    def _cm_init():
        cm_ref[...] = cm_blk

    @pl.when(qb > 0)
    def _cm_acc():
        cm_ref[...] = jnp.maximum(cm_ref[...], cm_blk)

    ov = jax.lax.dot_general(
        att, vt_ref[...], (((1,), (1,)), ((), ())),
        preferred_element_type=jnp.float32)
    out_ref[0] = jax.lax.dot_general(
        ov, wo_ref[...], (((1,), (1,)), ((), ())),
        preferred_element_type=jnp.float32,
        precision=jax.lax.Precision.HIGHEST) + bo_ref[...]

    @pl.when(qb == NQB - 1)
    def _voxel():
        cm = cm_ref[...]
        mn = jnp.min(cm)
        mx = jnp.max(cm)
        norm = (cm - mn) / (mx - mn)
        voxel = VOXEL_BASE + (1.0 - norm) * VOXEL_RANGE
        voxel_ref[0] = voxel
        vmn = jnp.min(voxel)
        vmx = jnp.max(voxel)

        @pl.when(n == 0)
        def _g_init():
            gmm_ref[0] = vmn
            gmm_ref[1] = vmx

        @pl.when(n > 0)
        def _g_acc():
            gmm_ref[0] = jnp.minimum(gmm_ref[0], vmn)
            gmm_ref[1] = jnp.maximum(gmm_ref[1], vmx)

        @pl.when(n == N - 1)
        def _bins():
            i_f = lax.broadcasted_iota(jnp.int32, (1, 128), 1).astype(jnp.float32)
            step = i_f / np.float32(BIN_SIZE)
            bins_ref[...] = gmm_ref[0] * (1.0 - step) + gmm_ref[1] * step


def _attention_call(batched_pts, Wv, Wk, Wq, Wo, bo2, interpret=False):
    return pl.pallas_call(
        _attn_body,
        grid=(N, NQB),
        in_specs=[
            pl.BlockSpec((1, D, P), lambda n, q: (n, 0, 0)),
            pl.BlockSpec((D, D), lambda n, q: (0, 0)),
            pl.BlockSpec((D, D), lambda n, q: (0, 0)),
            pl.BlockSpec((D, D), lambda n, q: (0, 0)),
            pl.BlockSpec((D, D), lambda n, q: (0, 0)),
            pl.BlockSpec((1, D), lambda n, q: (0, 0)),
        ],
        out_specs=[
            pl.BlockSpec((1, BQ, D), lambda n, q: (n, q, 0)),
            pl.BlockSpec((1, 1, P), lambda n, q: (n, 0, 0)),
            pl.BlockSpec((1, 128), lambda n, q: (0, 0)),
        ],
        out_shape=[
            jax.ShapeDtypeStruct((N, P, D), jnp.float32),
            jax.ShapeDtypeStruct((N, 1, P), jnp.float32),
            jax.ShapeDtypeStruct((1, 128), jnp.float32),
        ],
        scratch_shapes=[
            pltpu.VMEM((D, P), jnp.bfloat16),
            pltpu.VMEM((D, P), jnp.bfloat16),
            pltpu.VMEM((D, P), jnp.float32),
            pltpu.VMEM((1, P), jnp.float32),
            pltpu.SMEM((2,), jnp.float32),
        ],
        interpret=interpret,
    )(batched_pts, Wv, Wk, Wq, Wo, bo2)


def kernel(batched_pts, Wv, Wk, Wq, Wo, bo):
    out, voxel3, bins128 = _attention_call(
        batched_pts, Wv, Wk, Wq, Wo, bo.reshape(1, D))
    voxel_sizes = voxel3.reshape(N, P)

    bins = bins128[0, :BIN_SIZE + 1]
    flat_vs = voxel_sizes.reshape(-1)
    bin_index = jnp.searchsorted(bins, flat_vs, side='right') - 1
    bin_index = jnp.where((bin_index < 0) | (bin_index >= BIN_SIZE),
                          BIN_SIZE - 1, bin_index)
    counts = jnp.bincount(bin_index, length=BIN_SIZE)
    xyz_flat = jnp.transpose(batched_pts, (0, 2, 1)).reshape(-1, D)
    bin_sums = jax.ops.segment_sum(xyz_flat, bin_index, num_segments=BIN_SIZE)
    return out, voxel_sizes, counts, bin_sums


# speedup vs baseline: 1.8180x; 1.1598x over previous
import functools

import jax
import jax.numpy as jnp
import numpy as np
from jax import lax
from jax.experimental import pallas as pl
from jax.experimental.pallas import tpu as pltpu
from jax.experimental.pallas import tpu_sc as plsc

N, D, P, H = 8, 32, 4096, 1
HD = D // H
VOXEL_BASE = 0.05
VOXEL_RANGE = 0.1
BIN_SIZE = 10

BQ = 512
NQB = P // BQ
_INV_SQRT_D = np.float32(0.176776692)


def _attn_body(pts_ref, wv_ref, wk_ref, wq_ref, wo_ref, bo_ref,
               out_ref, voxel_ref, bins_ref,
               qt_ref, kt_ref, vt_ref, cm_ref, gmm_ref):
    n = pl.program_id(0)
    qb = pl.program_id(1)
    pts = pts_ref[0]

    @pl.when(qb == 0)
    def _project():
        ptsb = pts.astype(jnp.bfloat16)
        qt_ref[...] = jax.lax.dot_general(
            wq_ref[...].astype(jnp.bfloat16), ptsb, (((1,), (0,)), ((), ())),
            preferred_element_type=jnp.float32).astype(jnp.bfloat16)
        kt_ref[...] = jax.lax.dot_general(
            wk_ref[...].astype(jnp.bfloat16), ptsb, (((1,), (0,)), ((), ())),
            preferred_element_type=jnp.float32).astype(jnp.bfloat16)
        vt_ref[...] = jax.lax.dot_general(
            wv_ref[...], pts, (((1,), (0,)), ((), ())),
            preferred_element_type=jnp.float32,
            precision=jax.lax.Precision.HIGHEST)

    q_blk = qt_ref[:, pl.ds(qb * BQ, BQ)]
    e = jax.lax.dot_general(
        q_blk, kt_ref[...], (((0,), (0,)), ((), ())),
        preferred_element_type=jnp.float32) * _INV_SQRT_D
    m = jnp.max(e, axis=1, keepdims=True)
    p = jnp.exp(e - m)
    l = jnp.sum(p, axis=1, keepdims=True)
    att = p * (1.0 / l)

    cm_blk = jnp.max(att, axis=0, keepdims=True)

    @pl.when(qb == 0)
    def _cm_init():
        cm_ref[...] = cm_blk

    @pl.when(qb > 0)
    def _cm_acc():
        cm_ref[...] = jnp.maximum(cm_ref[...], cm_blk)

    ov = jax.lax.dot_general(
        att, vt_ref[...], (((1,), (1,)), ((), ())),
        preferred_element_type=jnp.float32)
    out_ref[0] = jax.lax.dot_general(
        ov, wo_ref[...], (((1,), (1,)), ((), ())),
        preferred_element_type=jnp.float32,
        precision=jax.lax.Precision.HIGHEST) + bo_ref[...]

    @pl.when(qb == NQB - 1)
    def _voxel():
        cm = cm_ref[...]
        mn = jnp.min(cm)
        mx = jnp.max(cm)
        norm = (cm - mn) / (mx - mn)
        voxel = VOXEL_BASE + (1.0 - norm) * VOXEL_RANGE
        voxel_ref[0] = voxel
        vmn = jnp.min(voxel)
        vmx = jnp.max(voxel)

        @pl.when(n == 0)
        def _g_init():
            gmm_ref[0] = vmn
            gmm_ref[1] = vmx

        @pl.when(n > 0)
        def _g_acc():
            gmm_ref[0] = jnp.minimum(gmm_ref[0], vmn)
            gmm_ref[1] = jnp.maximum(gmm_ref[1], vmx)

        @pl.when(n == N - 1)
        def _bins():
            lane = lax.broadcasted_iota(jnp.int32, (1, 256), 1)
            i_f = (lane // 16).astype(jnp.float32)
            step = i_f / np.float32(BIN_SIZE)
            bins_ref[...] = gmm_ref[0] * (1.0 - step) + gmm_ref[1] * step


def _attention_call(batched_pts, Wv, Wk, Wq, Wo, bo2, interpret=False):
    return pl.pallas_call(
        _attn_body,
        grid=(N, NQB),
        in_specs=[
            pl.BlockSpec((1, D, P), lambda n, q: (n, 0, 0)),
            pl.BlockSpec((D, D), lambda n, q: (0, 0)),
            pl.BlockSpec((D, D), lambda n, q: (0, 0)),
            pl.BlockSpec((D, D), lambda n, q: (0, 0)),
            pl.BlockSpec((D, D), lambda n, q: (0, 0)),
            pl.BlockSpec((1, D), lambda n, q: (0, 0)),
        ],
        out_specs=[
            pl.BlockSpec((1, BQ, D), lambda n, q: (n, q, 0)),
            pl.BlockSpec((1, 1, P), lambda n, q: (n, 0, 0)),
            pl.BlockSpec((1, 256), lambda n, q: (0, 0)),
        ],
        out_shape=[
            jax.ShapeDtypeStruct((N, P, D), jnp.float32),
            jax.ShapeDtypeStruct((N, 1, P), jnp.float32),
            jax.ShapeDtypeStruct((1, 256), jnp.float32),
        ],
        scratch_shapes=[
            pltpu.VMEM((D, P), jnp.bfloat16),
            pltpu.VMEM((D, P), jnp.bfloat16),
            pltpu.VMEM((D, P), jnp.float32),
            pltpu.VMEM((1, P), jnp.float32),
            pltpu.SMEM((2,), jnp.float32),
        ],
        interpret=interpret,
    )(batched_pts, Wv, Wk, Wq, Wo, bo2)


NW = 32
CHUNK = P * N // NW
NG = CHUNK // 16
LHIST = BIN_SIZE * D


def _sc_hist_body(pts_hbm, vox_hbm, bins_hbm, psums_hbm, pcnts_hbm,
                  pts_v, vox_v, bins_v, hist_v, cnt_v, psum_v, pcnt_v):
    wid = lax.axis_index("c") * 16 + lax.axis_index("s")
    n = wid // 4
    off = (wid % 4) * CHUNK

    pltpu.sync_copy(bins_hbm, bins_v)
    pltpu.sync_copy(vox_hbm.at[n, pl.ds(off, CHUNK)], vox_v)
    for dd in range(D):
        pltpu.sync_copy(pts_hbm.at[n, dd, pl.ds(off, CHUNK)], pts_v.at[dd])

    zeros = jnp.zeros((16,), jnp.float32)
    zeros_i = jnp.zeros((16,), jnp.int32)
    ones_iv = jnp.full((16,), 1, jnp.int32)
    nine_iv = jnp.full((16,), BIN_SIZE - 1, jnp.int32)
    ten_iv = jnp.full((16,), BIN_SIZE, jnp.int32)
    dim_iv = jnp.full((16,), D, jnp.int32)
    iota = lax.iota(jnp.int32, 16)
    lane_hist = iota * jnp.full((16,), LHIST, jnp.int32)
    lane_cnt = iota * jnp.full((16,), 16, jnp.int32)

    ones_fv = jnp.full((16,), 1.0, jnp.float32)
    for j in range(16 * LHIST // 16):
        hist_v[pl.ds(j * 16, 16)] = zeros
    for j in range(16):
        cnt_v[pl.ds(j * 16, 16)] = zeros

    bcast_bins = [bins_v[pl.ds(i * 16, 16)] for i in range(BIN_SIZE + 1)]

    def group(g, carry):
        v = vox_v[pl.ds(g * 16, 16)]
        c = lax.select(bcast_bins[0] <= v, ones_iv, zeros_i)
        for i in range(1, BIN_SIZE + 1):
            c = c + lax.select(bcast_bins[i] <= v, ones_iv, zeros_i)
        b = c - ones_iv
        bad = jnp.logical_or(b < zeros_i, b >= ten_iv)
        b = lax.select(bad, nine_iv, b)
        plsc.addupdate_scatter(cnt_v, [lane_cnt + b], ones_fv)
        idx = lane_hist + b * dim_iv
        for dd in range(D):
            feat = pts_v[dd, pl.ds(g * 16, 16)]
            plsc.addupdate_scatter(hist_v, [idx], feat)
            idx = idx + ones_iv
        return carry

    lax.fori_loop(0, NG, group, 0)

    for c in range(LHIST // 16):
        acc = hist_v[pl.ds(c * 16, 16)]
        for ln in range(1, 16):
            acc = acc + hist_v[pl.ds(ln * LHIST + c * 16, 16)]
        psum_v[pl.ds(c * 16, 16)] = acc
    cacc = cnt_v[pl.ds(0, 16)]
    for ln in range(1, 16):
        cacc = cacc + cnt_v[pl.ds(ln * 16, 16)]
    pcnt_v[...] = cacc.astype(jnp.int32)

    pltpu.sync_copy(psum_v, psums_hbm.at[wid])
    pltpu.sync_copy(pcnt_v, pcnts_hbm.at[wid])


def _sc_hist_call(batched_pts, voxel_sizes, bins):
    f = pl.kernel(
        _sc_hist_body,
        out_type=[jax.ShapeDtypeStruct((NW, LHIST), jnp.float32),
                  jax.ShapeDtypeStruct((NW, 16), jnp.int32)],
        mesh=plsc.VectorSubcoreMesh(core_axis_name="c", subcore_axis_name="s"),
        compiler_params=pltpu.CompilerParams(needs_layout_passes=False),
        scratch_types=[
            pltpu.VMEM((D, CHUNK), jnp.float32),
            pltpu.VMEM((CHUNK,), jnp.float32),
            pltpu.VMEM((256,), jnp.float32),
            pltpu.VMEM((16 * LHIST,), jnp.float32),
            pltpu.VMEM((16 * 16,), jnp.float32),
            pltpu.VMEM((LHIST,), jnp.float32),
            pltpu.VMEM((16,), jnp.int32),
        ],
    )
    return f(batched_pts, voxel_sizes, bins)


def _combine_body(ps_ref, pc_ref, s_ref, c_ref):
    s_ref[...] = jnp.sum(ps_ref[...], axis=0, keepdims=True)
    c_ref[...] = jnp.sum(pc_ref[...], axis=0, keepdims=True)


def _combine_call(psums, pcnts):
    return pl.pallas_call(
        _combine_body,
        out_shape=[jax.ShapeDtypeStruct((1, LHIST), jnp.float32),
                   jax.ShapeDtypeStruct((1, 16), jnp.int32)],
    )(psums, pcnts)


def kernel(batched_pts, Wv, Wk, Wq, Wo, bo):
    out, voxel3, bins128 = _attention_call(
        batched_pts, Wv, Wk, Wq, Wo, bo.reshape(1, D))
    voxel_sizes = voxel3.reshape(N, P)
    psums, pcnts = _sc_hist_call(batched_pts, voxel_sizes,
                                 bins128.reshape(256))
    s, c = _combine_call(psums, pcnts)
    bin_sums = s.reshape(BIN_SIZE, D)
    counts = c.reshape(16)[:BIN_SIZE]
    return out, voxel_sizes, counts, bin_sums


# speedup vs baseline: 1.8850x; 1.0368x over previous
import functools

import jax
import jax.numpy as jnp
import numpy as np
from jax import lax
from jax.experimental import pallas as pl
from jax.experimental.pallas import tpu as pltpu
from jax.experimental.pallas import tpu_sc as plsc

N, D, P, H = 8, 32, 4096, 1
HD = D // H
VOXEL_BASE = 0.05
VOXEL_RANGE = 0.1
BIN_SIZE = 10

BQ = 512
NQB = P // BQ
_INV_SQRT_D = np.float32(0.176776692)


def _attn_body(pts_ref, wv_ref, wk_ref, wq_ref, wo_ref, bo_ref,
               out_ref, voxel_ref, bins_ref,
               qt_ref, kt_ref, vt_ref, cm_ref, gmm_ref):
    n = pl.program_id(0)
    qb = pl.program_id(1)
    pts = pts_ref[0]

    @pl.when(qb == 0)
    def _project():
        ptsb = pts.astype(jnp.bfloat16)
        qt_ref[...] = jax.lax.dot_general(
            wq_ref[...].astype(jnp.bfloat16), ptsb, (((1,), (0,)), ((), ())),
            preferred_element_type=jnp.float32).astype(jnp.bfloat16)
        kt_ref[...] = jax.lax.dot_general(
            wk_ref[...].astype(jnp.bfloat16), ptsb, (((1,), (0,)), ((), ())),
            preferred_element_type=jnp.float32).astype(jnp.bfloat16)
        vt_ref[...] = jax.lax.dot_general(
            wv_ref[...], pts, (((1,), (0,)), ((), ())),
            preferred_element_type=jnp.float32,
            precision=jax.lax.Precision.HIGHEST)

    q_blk = qt_ref[:, pl.ds(qb * BQ, BQ)]
    e = jax.lax.dot_general(
        q_blk, kt_ref[...], (((0,), (0,)), ((), ())),
        preferred_element_type=jnp.float32) * _INV_SQRT_D
    m = jnp.max(e, axis=1, keepdims=True)
    p = jnp.exp(e - m)
    l = jnp.sum(p, axis=1, keepdims=True)
    att = p * (1.0 / l)

    cm_blk = jnp.max(att, axis=0, keepdims=True)

    @pl.when(qb == 0)
    def _cm_init():
        cm_ref[...] = cm_blk

    @pl.when(qb > 0)
    def _cm_acc():
        cm_ref[...] = jnp.maximum(cm_ref[...], cm_blk)

    ov = jax.lax.dot_general(
        att, vt_ref[...], (((1,), (1,)), ((), ())),
        preferred_element_type=jnp.float32)
    out_ref[0] = jax.lax.dot_general(
        ov, wo_ref[...], (((1,), (1,)), ((), ())),
        preferred_element_type=jnp.float32,
        precision=jax.lax.Precision.HIGHEST) + bo_ref[...]

    @pl.when(qb == NQB - 1)
    def _voxel():
        cm = cm_ref[...]
        mn = jnp.min(cm)
        mx = jnp.max(cm)
        norm = (cm - mn) / (mx - mn)
        voxel = VOXEL_BASE + (1.0 - norm) * VOXEL_RANGE
        voxel_ref[0] = voxel
        vmn = jnp.min(voxel)
        vmx = jnp.max(voxel)

        @pl.when(n == 0)
        def _g_init():
            gmm_ref[0] = vmn
            gmm_ref[1] = vmx

        @pl.when(n > 0)
        def _g_acc():
            gmm_ref[0] = jnp.minimum(gmm_ref[0], vmn)
            gmm_ref[1] = jnp.maximum(gmm_ref[1], vmx)

        @pl.when(n == N - 1)
        def _bins():
            lane = lax.broadcasted_iota(jnp.int32, (1, 256), 1)
            i_f = (lane // 16).astype(jnp.float32)
            step = i_f / np.float32(BIN_SIZE)
            bins_ref[...] = gmm_ref[0] * (1.0 - step) + gmm_ref[1] * step


def _attention_call(batched_pts, Wv, Wk, Wq, Wo, bo2, interpret=False):
    return pl.pallas_call(
        _attn_body,
        grid=(N, NQB),
        in_specs=[
            pl.BlockSpec((1, D, P), lambda n, q: (n, 0, 0)),
            pl.BlockSpec((D, D), lambda n, q: (0, 0)),
            pl.BlockSpec((D, D), lambda n, q: (0, 0)),
            pl.BlockSpec((D, D), lambda n, q: (0, 0)),
            pl.BlockSpec((D, D), lambda n, q: (0, 0)),
            pl.BlockSpec((1, D), lambda n, q: (0, 0)),
        ],
        out_specs=[
            pl.BlockSpec((1, BQ, D), lambda n, q: (n, q, 0)),
            pl.BlockSpec((1, 1, P), lambda n, q: (n, 0, 0)),
            pl.BlockSpec((1, 256), lambda n, q: (0, 0)),
        ],
        out_shape=[
            jax.ShapeDtypeStruct((N, P, D), jnp.float32),
            jax.ShapeDtypeStruct((N, 1, P), jnp.float32),
            jax.ShapeDtypeStruct((1, 256), jnp.float32),
        ],
        scratch_shapes=[
            pltpu.VMEM((D, P), jnp.bfloat16),
            pltpu.VMEM((D, P), jnp.bfloat16),
            pltpu.VMEM((D, P), jnp.float32),
            pltpu.VMEM((1, P), jnp.float32),
            pltpu.SMEM((2,), jnp.float32),
        ],
        interpret=interpret,
    )(batched_pts, Wv, Wk, Wq, Wo, bo2)


NW = 32
CHUNK = P * N // NW
NG = CHUNK // 16
LHIST = BIN_SIZE * D


def _sc_hist_body(pts_hbm, vox_hbm, bins_hbm, psums_hbm, pcnts_hbm,
                  pts_v, vox_v, bins_v, hist_v, cnt_v, psum_v, pcnt_v):
    wid = lax.axis_index("c") * 16 + lax.axis_index("s")
    n = wid // 4
    off = (wid % 4) * CHUNK

    pltpu.sync_copy(bins_hbm, bins_v)
    pltpu.sync_copy(vox_hbm.at[n, pl.ds(off, CHUNK)], vox_v)
    pltpu.sync_copy(pts_hbm.at[n, :, pl.ds(off, CHUNK)], pts_v)

    zeros = jnp.zeros((16,), jnp.float32)
    zeros_i = jnp.zeros((16,), jnp.int32)
    ones_iv = jnp.full((16,), 1, jnp.int32)
    nine_iv = jnp.full((16,), BIN_SIZE - 1, jnp.int32)
    ten_iv = jnp.full((16,), BIN_SIZE, jnp.int32)
    dim_iv = jnp.full((16,), D, jnp.int32)
    iota = lax.iota(jnp.int32, 16)
    lane_hist = iota * jnp.full((16,), LHIST, jnp.int32)
    lane_cnt = iota * jnp.full((16,), 16, jnp.int32)

    ones_fv = jnp.full((16,), 1.0, jnp.float32)
    for j in range(16 * LHIST // 16):
        hist_v[pl.ds(j * 16, 16)] = zeros
    for j in range(16):
        cnt_v[pl.ds(j * 16, 16)] = zeros

    bcast_bins = [bins_v[pl.ds(i * 16, 16)] for i in range(BIN_SIZE + 1)]

    def group(g, carry):
        v = vox_v[pl.ds(g * 16, 16)]
        c = lax.select(bcast_bins[0] <= v, ones_iv, zeros_i)
        for i in range(1, BIN_SIZE + 1):
            c = c + lax.select(bcast_bins[i] <= v, ones_iv, zeros_i)
        b = c - ones_iv
        bad = jnp.logical_or(b < zeros_i, b >= ten_iv)
        b = lax.select(bad, nine_iv, b)
        plsc.addupdate_scatter(cnt_v, [lane_cnt + b], ones_fv)
        idx = lane_hist + b * dim_iv
        for dd in range(D):
            feat = pts_v[dd, pl.ds(g * 16, 16)]
            plsc.addupdate_scatter(hist_v, [idx], feat)
            idx = idx + ones_iv
        return carry

    lax.fori_loop(0, NG, group, 0)

    for c in range(LHIST // 16):
        acc = hist_v[pl.ds(c * 16, 16)]
        for ln in range(1, 16):
            acc = acc + hist_v[pl.ds(ln * LHIST + c * 16, 16)]
        psum_v[pl.ds(c * 16, 16)] = acc
    cacc = cnt_v[pl.ds(0, 16)]
    for ln in range(1, 16):
        cacc = cacc + cnt_v[pl.ds(ln * 16, 16)]
    pcnt_v[...] = cacc.astype(jnp.int32)

    pltpu.sync_copy(psum_v, psums_hbm.at[wid])
    pltpu.sync_copy(pcnt_v, pcnts_hbm.at[wid])


def _sc_hist_call(batched_pts, voxel_sizes, bins):
    f = pl.kernel(
        _sc_hist_body,
        out_type=[jax.ShapeDtypeStruct((NW, LHIST), jnp.float32),
                  jax.ShapeDtypeStruct((NW, 16), jnp.int32)],
        mesh=plsc.VectorSubcoreMesh(core_axis_name="c", subcore_axis_name="s"),
        compiler_params=pltpu.CompilerParams(needs_layout_passes=False),
        scratch_types=[
            pltpu.VMEM((D, CHUNK), jnp.float32),
            pltpu.VMEM((CHUNK,), jnp.float32),
            pltpu.VMEM((256,), jnp.float32),
            pltpu.VMEM((16 * LHIST,), jnp.float32),
            pltpu.VMEM((16 * 16,), jnp.float32),
            pltpu.VMEM((LHIST,), jnp.float32),
            pltpu.VMEM((16,), jnp.int32),
        ],
    )
    return f(batched_pts, voxel_sizes, bins)


def _combine_body(ps_ref, pc_ref, s_ref, c_ref):
    s_ref[...] = jnp.sum(ps_ref[...], axis=0, keepdims=True)
    c_ref[...] = jnp.sum(pc_ref[...], axis=0, keepdims=True)


def _combine_call(psums, pcnts):
    return pl.pallas_call(
        _combine_body,
        out_shape=[jax.ShapeDtypeStruct((1, LHIST), jnp.float32),
                   jax.ShapeDtypeStruct((1, 16), jnp.int32)],
    )(psums, pcnts)


def kernel(batched_pts, Wv, Wk, Wq, Wo, bo):
    out, voxel3, bins128 = _attention_call(
        batched_pts, Wv, Wk, Wq, Wo, bo.reshape(1, D))
    voxel_sizes = voxel3.reshape(N, P)
    psums, pcnts = _sc_hist_call(batched_pts, voxel_sizes,
                                 bins128.reshape(256))
    s, c = _combine_call(psums, pcnts)
    bin_sums = s.reshape(BIN_SIZE, D)
    counts = c.reshape(16)[:BIN_SIZE]
    return out, voxel_sizes, counts, bin_sums


# speedup vs baseline: 1.9636x; 1.0417x over previous
import functools

import jax
import jax.numpy as jnp
import numpy as np
from jax import lax
from jax.experimental import pallas as pl
from jax.experimental.pallas import tpu as pltpu
from jax.experimental.pallas import tpu_sc as plsc

N, D, P, H = 8, 32, 4096, 1
HD = D // H
VOXEL_BASE = 0.05
VOXEL_RANGE = 0.1
BIN_SIZE = 10

BQ = 1024
NQB = P // BQ
_INV_SQRT_D = np.float32(0.176776692)


def _attn_body(pts_ref, wv_ref, wk_ref, wq_ref, wo_ref, bo_ref,
               out_ref, voxel_ref, bins_ref,
               qt_ref, kt_ref, vt_ref, cm_ref, gmm_ref):
    n = pl.program_id(0)
    qb = pl.program_id(1)
    pts = pts_ref[0]

    @pl.when(qb == 0)
    def _project():
        ptsb = pts.astype(jnp.bfloat16)
        qt_ref[...] = jax.lax.dot_general(
            wq_ref[...].astype(jnp.bfloat16), ptsb, (((1,), (0,)), ((), ())),
            preferred_element_type=jnp.float32).astype(jnp.bfloat16)
        kt_ref[...] = jax.lax.dot_general(
            wk_ref[...].astype(jnp.bfloat16), ptsb, (((1,), (0,)), ((), ())),
            preferred_element_type=jnp.float32).astype(jnp.bfloat16)
        vt_ref[...] = jax.lax.dot_general(
            wv_ref[...], pts, (((1,), (0,)), ((), ())),
            preferred_element_type=jnp.float32,
            precision=jax.lax.Precision.HIGHEST)

    q_blk = qt_ref[:, pl.ds(qb * BQ, BQ)]
    e = jax.lax.dot_general(
        q_blk, kt_ref[...], (((0,), (0,)), ((), ())),
        preferred_element_type=jnp.float32) * _INV_SQRT_D
    m = jnp.max(e, axis=1, keepdims=True)
    p = jnp.exp(e - m)
    l = jnp.sum(p, axis=1, keepdims=True)
    att = p * (1.0 / l)

    cm_blk = jnp.max(att, axis=0, keepdims=True)

    @pl.when(qb == 0)
    def _cm_init():
        cm_ref[...] = cm_blk

    @pl.when(qb > 0)
    def _cm_acc():
        cm_ref[...] = jnp.maximum(cm_ref[...], cm_blk)

    ov = jax.lax.dot_general(
        att, vt_ref[...], (((1,), (1,)), ((), ())),
        preferred_element_type=jnp.float32)
    out_ref[0] = jax.lax.dot_general(
        ov, wo_ref[...], (((1,), (1,)), ((), ())),
        preferred_element_type=jnp.float32,
        precision=jax.lax.Precision.HIGHEST) + bo_ref[...]

    @pl.when(qb == NQB - 1)
    def _voxel():
        cm = cm_ref[...]
        mn = jnp.min(cm)
        mx = jnp.max(cm)
        norm = (cm - mn) / (mx - mn)
        voxel = VOXEL_BASE + (1.0 - norm) * VOXEL_RANGE
        voxel_ref[0] = voxel
        vmn = jnp.min(voxel)
        vmx = jnp.max(voxel)

        @pl.when(n == 0)
        def _g_init():
            gmm_ref[0] = vmn
            gmm_ref[1] = vmx

        @pl.when(n > 0)
        def _g_acc():
            gmm_ref[0] = jnp.minimum(gmm_ref[0], vmn)
            gmm_ref[1] = jnp.maximum(gmm_ref[1], vmx)

        @pl.when(n == N - 1)
        def _bins():
            lane = lax.broadcasted_iota(jnp.int32, (1, 256), 1)
            i_f = (lane // 16).astype(jnp.float32)
            step = i_f / np.float32(BIN_SIZE)
            bins_ref[...] = gmm_ref[0] * (1.0 - step) + gmm_ref[1] * step


def _attention_call(batched_pts, Wv, Wk, Wq, Wo, bo2, interpret=False):
    return pl.pallas_call(
        _attn_body,
        grid=(N, NQB),
        in_specs=[
            pl.BlockSpec((1, D, P), lambda n, q: (n, 0, 0)),
            pl.BlockSpec((D, D), lambda n, q: (0, 0)),
            pl.BlockSpec((D, D), lambda n, q: (0, 0)),
            pl.BlockSpec((D, D), lambda n, q: (0, 0)),
            pl.BlockSpec((D, D), lambda n, q: (0, 0)),
            pl.BlockSpec((1, D), lambda n, q: (0, 0)),
        ],
        out_specs=[
            pl.BlockSpec((1, BQ, D), lambda n, q: (n, q, 0)),
            pl.BlockSpec((1, 1, P), lambda n, q: (n, 0, 0)),
            pl.BlockSpec((1, 256), lambda n, q: (0, 0)),
        ],
        out_shape=[
            jax.ShapeDtypeStruct((N, P, D), jnp.float32),
            jax.ShapeDtypeStruct((N, 1, P), jnp.float32),
            jax.ShapeDtypeStruct((1, 256), jnp.float32),
        ],
        scratch_shapes=[
            pltpu.VMEM((D, P), jnp.bfloat16),
            pltpu.VMEM((D, P), jnp.bfloat16),
            pltpu.VMEM((D, P), jnp.float32),
            pltpu.VMEM((1, P), jnp.float32),
            pltpu.SMEM((2,), jnp.float32),
        ],
        interpret=interpret,
    )(batched_pts, Wv, Wk, Wq, Wo, bo2)


NW = 32
CHUNK = P * N // NW
NG = CHUNK // 16
LHIST = BIN_SIZE * D


def _sc_hist_body(pts_hbm, vox_hbm, bins_hbm, psums_hbm, pcnts_hbm,
                  pts_v, vox_v, bins_v, hist_v, cnt_v, psum_v, pcnt_v):
    wid = lax.axis_index("c") * 16 + lax.axis_index("s")
    n = wid // 4
    off = (wid % 4) * CHUNK

    pltpu.sync_copy(bins_hbm, bins_v)
    pltpu.sync_copy(vox_hbm.at[n, pl.ds(off, CHUNK)], vox_v)
    pltpu.sync_copy(pts_hbm.at[n, :, pl.ds(off, CHUNK)], pts_v)

    zeros = jnp.zeros((16,), jnp.float32)
    zeros_i = jnp.zeros((16,), jnp.int32)
    ones_iv = jnp.full((16,), 1, jnp.int32)
    nine_iv = jnp.full((16,), BIN_SIZE - 1, jnp.int32)
    ten_iv = jnp.full((16,), BIN_SIZE, jnp.int32)
    dim_iv = jnp.full((16,), D, jnp.int32)
    iota = lax.iota(jnp.int32, 16)
    lane_hist = iota * jnp.full((16,), LHIST, jnp.int32)
    lane_cnt = iota * jnp.full((16,), 16, jnp.int32)

    ones_fv = jnp.full((16,), 1.0, jnp.float32)
    for j in range(16 * LHIST // 16):
        hist_v[pl.ds(j * 16, 16)] = zeros
    for j in range(16):
        cnt_v[pl.ds(j * 16, 16)] = zeros

    bcast_bins = [bins_v[pl.ds(i * 16, 16)] for i in range(BIN_SIZE + 1)]

    def group(g, carry):
        v = vox_v[pl.ds(g * 16, 16)]
        c = lax.select(bcast_bins[0] <= v, ones_iv, zeros_i)
        for i in range(1, BIN_SIZE + 1):
            c = c + lax.select(bcast_bins[i] <= v, ones_iv, zeros_i)
        b = c - ones_iv
        bad = jnp.logical_or(b < zeros_i, b >= ten_iv)
        b = lax.select(bad, nine_iv, b)
        plsc.addupdate_scatter(cnt_v, [lane_cnt + b], ones_fv)
        idx = lane_hist + b * dim_iv
        for dd in range(D):
            feat = pts_v[dd, pl.ds(g * 16, 16)]
            plsc.addupdate_scatter(hist_v, [idx], feat)
            idx = idx + ones_iv
        return carry

    lax.fori_loop(0, NG, group, 0)

    for c in range(LHIST // 16):
        acc = hist_v[pl.ds(c * 16, 16)]
        for ln in range(1, 16):
            acc = acc + hist_v[pl.ds(ln * LHIST + c * 16, 16)]
        psum_v[pl.ds(c * 16, 16)] = acc
    cacc = cnt_v[pl.ds(0, 16)]
    for ln in range(1, 16):
        cacc = cacc + cnt_v[pl.ds(ln * 16, 16)]
    pcnt_v[...] = cacc.astype(jnp.int32)

    pltpu.sync_copy(psum_v, psums_hbm.at[wid])
    pltpu.sync_copy(pcnt_v, pcnts_hbm.at[wid])


def _sc_hist_call(batched_pts, voxel_sizes, bins):
    f = pl.kernel(
        _sc_hist_body,
        out_type=[jax.ShapeDtypeStruct((NW, LHIST), jnp.float32),
                  jax.ShapeDtypeStruct((NW, 16), jnp.int32)],
        mesh=plsc.VectorSubcoreMesh(core_axis_name="c", subcore_axis_name="s"),
        compiler_params=pltpu.CompilerParams(needs_layout_passes=False),
        scratch_types=[
            pltpu.VMEM((D, CHUNK), jnp.float32),
            pltpu.VMEM((CHUNK,), jnp.float32),
            pltpu.VMEM((256,), jnp.float32),
            pltpu.VMEM((16 * LHIST,), jnp.float32),
            pltpu.VMEM((16 * 16,), jnp.float32),
            pltpu.VMEM((LHIST,), jnp.float32),
            pltpu.VMEM((16,), jnp.int32),
        ],
    )
    return f(batched_pts, voxel_sizes, bins)


def _combine_body(ps_ref, pc_ref, s_ref, c_ref):
    s_ref[...] = jnp.sum(ps_ref[...], axis=0, keepdims=True)
    c_ref[...] = jnp.sum(pc_ref[...], axis=0, keepdims=True)


def _combine_call(psums, pcnts):
    return pl.pallas_call(
        _combine_body,
        out_shape=[jax.ShapeDtypeStruct((1, LHIST), jnp.float32),
                   jax.ShapeDtypeStruct((1, 16), jnp.int32)],
    )(psums, pcnts)


def kernel(batched_pts, Wv, Wk, Wq, Wo, bo):
    out, voxel3, bins128 = _attention_call(
        batched_pts, Wv, Wk, Wq, Wo, bo.reshape(1, D))
    voxel_sizes = voxel3.reshape(N, P)
    psums, pcnts = _sc_hist_call(batched_pts, voxel_sizes,
                                 bins128.reshape(256))
    s, c = _combine_call(psums, pcnts)
    bin_sums = s.reshape(BIN_SIZE, D)
    counts = c.reshape(16)[:BIN_SIZE]
    return out, voxel_sizes, counts, bin_sums


# speedup vs baseline: 1.9877x; 1.0123x over previous
import functools

import jax
import jax.numpy as jnp
import numpy as np
from jax import lax
from jax.experimental import pallas as pl
from jax.experimental.pallas import tpu as pltpu
from jax.experimental.pallas import tpu_sc as plsc

N, D, P, H = 8, 32, 4096, 1
HD = D // H
VOXEL_BASE = 0.05
VOXEL_RANGE = 0.1
BIN_SIZE = 10

BQ = 2048
NQB = P // BQ
_INV_SQRT_D = np.float32(0.176776692)


def _attn_body(pts_ref, wv_ref, wk_ref, wq_ref, wo_ref, bo_ref,
               out_ref, voxel_ref, bins_ref,
               qt_ref, kt_ref, vt_ref, cm_ref, gmm_ref):
    n = pl.program_id(0)
    qb = pl.program_id(1)
    pts = pts_ref[0]

    @pl.when(qb == 0)
    def _project():
        ptsb = pts.astype(jnp.bfloat16)
        qt_ref[...] = jax.lax.dot_general(
            wq_ref[...].astype(jnp.bfloat16), ptsb, (((1,), (0,)), ((), ())),
            preferred_element_type=jnp.float32).astype(jnp.bfloat16)
        kt_ref[...] = jax.lax.dot_general(
            wk_ref[...].astype(jnp.bfloat16), ptsb, (((1,), (0,)), ((), ())),
            preferred_element_type=jnp.float32).astype(jnp.bfloat16)
        vt_ref[...] = jax.lax.dot_general(
            wv_ref[...], pts, (((1,), (0,)), ((), ())),
            preferred_element_type=jnp.float32,
            precision=jax.lax.Precision.HIGHEST)

    q_blk = qt_ref[:, pl.ds(qb * BQ, BQ)]
    e = jax.lax.dot_general(
        q_blk, kt_ref[...], (((0,), (0,)), ((), ())),
        preferred_element_type=jnp.float32) * _INV_SQRT_D
    m = jnp.max(e, axis=1, keepdims=True)
    p = jnp.exp(e - m)
    l = jnp.sum(p, axis=1, keepdims=True)
    att = p * (1.0 / l)

    cm_blk = jnp.max(att, axis=0, keepdims=True)

    @pl.when(qb == 0)
    def _cm_init():
        cm_ref[...] = cm_blk

    @pl.when(qb > 0)
    def _cm_acc():
        cm_ref[...] = jnp.maximum(cm_ref[...], cm_blk)

    ov = jax.lax.dot_general(
        att, vt_ref[...], (((1,), (1,)), ((), ())),
        preferred_element_type=jnp.float32)
    out_ref[0] = jax.lax.dot_general(
        ov, wo_ref[...], (((1,), (1,)), ((), ())),
        preferred_element_type=jnp.float32,
        precision=jax.lax.Precision.HIGHEST) + bo_ref[...]

    @pl.when(qb == NQB - 1)
    def _voxel():
        cm = cm_ref[...]
        mn = jnp.min(cm)
        mx = jnp.max(cm)
        norm = (cm - mn) / (mx - mn)
        voxel = VOXEL_BASE + (1.0 - norm) * VOXEL_RANGE
        voxel_ref[0] = voxel
        vmn = jnp.min(voxel)
        vmx = jnp.max(voxel)

        @pl.when(n == 0)
        def _g_init():
            gmm_ref[0] = vmn
            gmm_ref[1] = vmx

        @pl.when(n > 0)
        def _g_acc():
            gmm_ref[0] = jnp.minimum(gmm_ref[0], vmn)
            gmm_ref[1] = jnp.maximum(gmm_ref[1], vmx)

        @pl.when(n == N - 1)
        def _bins():
            lane = lax.broadcasted_iota(jnp.int32, (1, 256), 1)
            i_f = (lane // 16).astype(jnp.float32)
            step = i_f / np.float32(BIN_SIZE)
            bins_ref[...] = gmm_ref[0] * (1.0 - step) + gmm_ref[1] * step


def _attention_call(batched_pts, Wv, Wk, Wq, Wo, bo2, interpret=False):
    return pl.pallas_call(
        _attn_body,
        grid=(N, NQB),
        in_specs=[
            pl.BlockSpec((1, D, P), lambda n, q: (n, 0, 0)),
            pl.BlockSpec((D, D), lambda n, q: (0, 0)),
            pl.BlockSpec((D, D), lambda n, q: (0, 0)),
            pl.BlockSpec((D, D), lambda n, q: (0, 0)),
            pl.BlockSpec((D, D), lambda n, q: (0, 0)),
            pl.BlockSpec((1, D), lambda n, q: (0, 0)),
        ],
        out_specs=[
            pl.BlockSpec((1, BQ, D), lambda n, q: (n, q, 0)),
            pl.BlockSpec((1, 1, P), lambda n, q: (n, 0, 0)),
            pl.BlockSpec((1, 256), lambda n, q: (0, 0)),
        ],
        out_shape=[
            jax.ShapeDtypeStruct((N, P, D), jnp.float32),
            jax.ShapeDtypeStruct((N, 1, P), jnp.float32),
            jax.ShapeDtypeStruct((1, 256), jnp.float32),
        ],
        scratch_shapes=[
            pltpu.VMEM((D, P), jnp.bfloat16),
            pltpu.VMEM((D, P), jnp.bfloat16),
            pltpu.VMEM((D, P), jnp.float32),
            pltpu.VMEM((1, P), jnp.float32),
            pltpu.SMEM((2,), jnp.float32),
        ],
        interpret=interpret,
    )(batched_pts, Wv, Wk, Wq, Wo, bo2)


NW = 32
CHUNK = P * N // NW
NG = CHUNK // 16
LHIST = BIN_SIZE * D


def _sc_hist_body(pts_hbm, vox_hbm, bins_hbm, psums_hbm, pcnts_hbm,
                  pts_v, vox_v, bins_v, hist_v, cnt_v, psum_v, pcnt_v):
    wid = lax.axis_index("c") * 16 + lax.axis_index("s")
    n = wid // 4
    off = (wid % 4) * CHUNK

    pltpu.sync_copy(bins_hbm, bins_v)
    pltpu.sync_copy(vox_hbm.at[n, pl.ds(off, CHUNK)], vox_v)
    pltpu.sync_copy(pts_hbm.at[n, :, pl.ds(off, CHUNK)], pts_v)

    zeros = jnp.zeros((16,), jnp.float32)
    zeros_i = jnp.zeros((16,), jnp.int32)
    ones_iv = jnp.full((16,), 1, jnp.int32)
    nine_iv = jnp.full((16,), BIN_SIZE - 1, jnp.int32)
    ten_iv = jnp.full((16,), BIN_SIZE, jnp.int32)
    dim_iv = jnp.full((16,), D, jnp.int32)
    iota = lax.iota(jnp.int32, 16)
    lane_hist = iota * jnp.full((16,), LHIST, jnp.int32)
    lane_cnt = iota * jnp.full((16,), 16, jnp.int32)

    ones_fv = jnp.full((16,), 1.0, jnp.float32)
    for j in range(16 * LHIST // 16):
        hist_v[pl.ds(j * 16, 16)] = zeros
    for j in range(16):
        cnt_v[pl.ds(j * 16, 16)] = zeros

    bcast_bins = [bins_v[pl.ds(i * 16, 16)] for i in range(BIN_SIZE + 1)]

    def group(g, carry):
        v = vox_v[pl.ds(g * 16, 16)]
        c = lax.select(bcast_bins[0] <= v, ones_iv, zeros_i)
        for i in range(1, BIN_SIZE + 1):
            c = c + lax.select(bcast_bins[i] <= v, ones_iv, zeros_i)
        b = c - ones_iv
        bad = jnp.logical_or(b < zeros_i, b >= ten_iv)
        b = lax.select(bad, nine_iv, b)
        plsc.addupdate_scatter(cnt_v, [lane_cnt + b], ones_fv)
        idx = lane_hist + b * dim_iv
        for dd in range(D):
            feat = pts_v[dd, pl.ds(g * 16, 16)]
            plsc.addupdate_scatter(hist_v, [idx], feat)
            idx = idx + ones_iv
        return carry

    lax.fori_loop(0, NG, group, 0)

    for c in range(LHIST // 16):
        acc = hist_v[pl.ds(c * 16, 16)]
        for ln in range(1, 16):
            acc = acc + hist_v[pl.ds(ln * LHIST + c * 16, 16)]
        psum_v[pl.ds(c * 16, 16)] = acc
    cacc = cnt_v[pl.ds(0, 16)]
    for ln in range(1, 16):
        cacc = cacc + cnt_v[pl.ds(ln * 16, 16)]
    pcnt_v[...] = cacc.astype(jnp.int32)

    pltpu.sync_copy(psum_v, psums_hbm.at[wid])
    pltpu.sync_copy(pcnt_v, pcnts_hbm.at[wid])


def _sc_hist_call(batched_pts, voxel_sizes, bins):
    f = pl.kernel(
        _sc_hist_body,
        out_type=[jax.ShapeDtypeStruct((NW, LHIST), jnp.float32),
                  jax.ShapeDtypeStruct((NW, 16), jnp.int32)],
        mesh=plsc.VectorSubcoreMesh(core_axis_name="c", subcore_axis_name="s"),
        compiler_params=pltpu.CompilerParams(needs_layout_passes=False),
        scratch_types=[
            pltpu.VMEM((D, CHUNK), jnp.float32),
            pltpu.VMEM((CHUNK,), jnp.float32),
            pltpu.VMEM((256,), jnp.float32),
            pltpu.VMEM((16 * LHIST,), jnp.float32),
            pltpu.VMEM((16 * 16,), jnp.float32),
            pltpu.VMEM((LHIST,), jnp.float32),
            pltpu.VMEM((16,), jnp.int32),
        ],
    )
    return f(batched_pts, voxel_sizes, bins)


def _combine_body(ps_ref, pc_ref, s_ref, c_ref):
    s_ref[...] = jnp.sum(ps_ref[...], axis=0, keepdims=True)
    c_ref[...] = jnp.sum(pc_ref[...], axis=0, keepdims=True)


def _combine_call(psums, pcnts):
    return pl.pallas_call(
        _combine_body,
        out_shape=[jax.ShapeDtypeStruct((1, LHIST), jnp.float32),
                   jax.ShapeDtypeStruct((1, 16), jnp.int32)],
    )(psums, pcnts)


def kernel(batched_pts, Wv, Wk, Wq, Wo, bo):
    out, voxel3, bins128 = _attention_call(
        batched_pts, Wv, Wk, Wq, Wo, bo.reshape(1, D))
    voxel_sizes = voxel3.reshape(N, P)
    psums, pcnts = _sc_hist_call(batched_pts, voxel_sizes,
                                 bins128.reshape(256))
    s, c = _combine_call(psums, pcnts)
    bin_sums = s.reshape(BIN_SIZE, D)
    counts = c.reshape(16)[:BIN_SIZE]
    return out, voxel_sizes, counts, bin_sums


# speedup vs baseline: 1.9998x; 1.0061x over previous
import functools

import jax
import jax.numpy as jnp
import numpy as np
from jax import lax
from jax.experimental import pallas as pl
from jax.experimental.pallas import tpu as pltpu
from jax.experimental.pallas import tpu_sc as plsc

N, D, P, H = 8, 32, 4096, 1
HD = D // H
VOXEL_BASE = 0.05
VOXEL_RANGE = 0.1
BIN_SIZE = 10

BQ = 2048
NQB = P // BQ
_INV_SQRT_D = np.float32(0.176776692)


def _attn_body(pts_ref, wv_ref, wk_ref, wq_ref, wo_ref, bo_ref,
               out_ref, voxel_ref, bins_ref,
               qt_ref, kt_ref, vt_ref, cm_ref, gmm_ref):
    n = pl.program_id(0)
    qb = pl.program_id(1)
    pts = pts_ref[0]

    @pl.when(qb == 0)
    def _project():
        ptsb = pts.astype(jnp.bfloat16)
        qt_ref[...] = jax.lax.dot_general(
            wq_ref[...].astype(jnp.bfloat16), ptsb, (((1,), (0,)), ((), ())),
            preferred_element_type=jnp.float32).astype(jnp.bfloat16)
        kt_ref[...] = jax.lax.dot_general(
            wk_ref[...].astype(jnp.bfloat16), ptsb, (((1,), (0,)), ((), ())),
            preferred_element_type=jnp.float32).astype(jnp.bfloat16)
        vt_ref[...] = jax.lax.dot_general(
            wv_ref[...], pts, (((1,), (0,)), ((), ())),
            preferred_element_type=jnp.float32,
            precision=jax.lax.Precision.HIGHEST)

    q_blk = qt_ref[:, pl.ds(qb * BQ, BQ)]
    e = jax.lax.dot_general(
        q_blk, kt_ref[...], (((0,), (0,)), ((), ())),
        preferred_element_type=jnp.float32) * _INV_SQRT_D
    m = jnp.max(e, axis=1, keepdims=True)
    p = jnp.exp(e - m)
    l = jnp.sum(p, axis=1, keepdims=True)
    att = p * (1.0 / l)

    cm_blk = jnp.max(att, axis=0, keepdims=True)

    @pl.when(qb == 0)
    def _cm_init():
        cm_ref[...] = cm_blk

    @pl.when(qb > 0)
    def _cm_acc():
        cm_ref[...] = jnp.maximum(cm_ref[...], cm_blk)

    ov = jax.lax.dot_general(
        att, vt_ref[...], (((1,), (1,)), ((), ())),
        preferred_element_type=jnp.float32)
    out_ref[0] = jax.lax.dot_general(
        ov, wo_ref[...], (((1,), (1,)), ((), ())),
        preferred_element_type=jnp.float32,
        precision=jax.lax.Precision.HIGHEST) + bo_ref[...]

    @pl.when(qb == NQB - 1)
    def _voxel():
        cm = cm_ref[...]
        mn = jnp.min(cm)
        mx = jnp.max(cm)
        norm = (cm - mn) / (mx - mn)
        voxel = VOXEL_BASE + (1.0 - norm) * VOXEL_RANGE
        voxel_ref[0] = voxel
        vmn = jnp.min(voxel)
        vmx = jnp.max(voxel)

        @pl.when(n == 0)
        def _g_init():
            gmm_ref[0] = vmn
            gmm_ref[1] = vmx

        @pl.when(n > 0)
        def _g_acc():
            gmm_ref[0] = jnp.minimum(gmm_ref[0], vmn)
            gmm_ref[1] = jnp.maximum(gmm_ref[1], vmx)

        @pl.when(n == N - 1)
        def _bins():
            lane = lax.broadcasted_iota(jnp.int32, (1, 256), 1)
            i_f = (lane // 16).astype(jnp.float32)
            step = i_f / np.float32(BIN_SIZE)
            bins_ref[...] = gmm_ref[0] * (1.0 - step) + gmm_ref[1] * step


def _attention_call(batched_pts, Wv, Wk, Wq, Wo, bo2, interpret=False):
    return pl.pallas_call(
        _attn_body,
        grid=(N, NQB),
        in_specs=[
            pl.BlockSpec((1, D, P), lambda n, q: (n, 0, 0)),
            pl.BlockSpec((D, D), lambda n, q: (0, 0)),
            pl.BlockSpec((D, D), lambda n, q: (0, 0)),
            pl.BlockSpec((D, D), lambda n, q: (0, 0)),
            pl.BlockSpec((D, D), lambda n, q: (0, 0)),
            pl.BlockSpec((1, D), lambda n, q: (0, 0)),
        ],
        out_specs=[
            pl.BlockSpec((1, BQ, D), lambda n, q: (n, q, 0)),
            pl.BlockSpec((1, 1, P), lambda n, q: (n, 0, 0)),
            pl.BlockSpec((1, 256), lambda n, q: (0, 0)),
        ],
        out_shape=[
            jax.ShapeDtypeStruct((N, P, D), jnp.float32),
            jax.ShapeDtypeStruct((N, 1, P), jnp.float32),
            jax.ShapeDtypeStruct((1, 256), jnp.float32),
        ],
        scratch_shapes=[
            pltpu.VMEM((D, P), jnp.bfloat16),
            pltpu.VMEM((D, P), jnp.bfloat16),
            pltpu.VMEM((D, P), jnp.float32),
            pltpu.VMEM((1, P), jnp.float32),
            pltpu.SMEM((2,), jnp.float32),
        ],
        interpret=interpret,
    )(batched_pts, Wv, Wk, Wq, Wo, bo2)


NW = 32
CHUNK = P * N // NW
NG = CHUNK // 16
LHIST = BIN_SIZE * D


def _sc_hist_body(pts_hbm, vox_hbm, bins_hbm, psums_hbm, pcnts_hbm,
                  pts_v, vox_v, bins_v, hist_v, cnt_v, psum_v, pcnt_v):
    wid = lax.axis_index("c") * 16 + lax.axis_index("s")
    n = wid // 4
    off = (wid % 4) * CHUNK

    pltpu.sync_copy(bins_hbm, bins_v)
    pltpu.sync_copy(vox_hbm.at[n, pl.ds(off, CHUNK)], vox_v)
    pltpu.sync_copy(pts_hbm.at[n, :, pl.ds(off, CHUNK)], pts_v)

    zeros = jnp.zeros((16,), jnp.float32)
    zeros_i = jnp.zeros((16,), jnp.int32)
    ones_iv = jnp.full((16,), 1, jnp.int32)
    nine_iv = jnp.full((16,), BIN_SIZE - 1, jnp.int32)
    ten_iv = jnp.full((16,), BIN_SIZE, jnp.int32)
    dim_iv = jnp.full((16,), D, jnp.int32)
    iota = lax.iota(jnp.int32, 16)
    lane_hist = iota * jnp.full((16,), LHIST, jnp.int32)
    lane_cnt = iota * jnp.full((16,), 16, jnp.int32)

    ones_fv = jnp.full((16,), 1.0, jnp.float32)
    for j in range(16 * LHIST // 16):
        hist_v[pl.ds(j * 16, 16)] = zeros
    for j in range(16):
        cnt_v[pl.ds(j * 16, 16)] = zeros

    bcast_bins = [bins_v[pl.ds(i * 16, 16)] for i in range(BIN_SIZE + 1)]

    def one_group(g):
        v = vox_v[pl.ds(g * 16, 16)]
        c = lax.select(bcast_bins[0] <= v, ones_iv, zeros_i)
        for i in range(1, BIN_SIZE + 1):
            c = c + lax.select(bcast_bins[i] <= v, ones_iv, zeros_i)
        b = c - ones_iv
        bad = jnp.logical_or(b < zeros_i, b >= ten_iv)
        b = lax.select(bad, nine_iv, b)
        plsc.addupdate_scatter(cnt_v, [lane_cnt + b], ones_fv)
        return lane_hist + b * dim_iv

    def group(g, carry):
        idx0 = one_group(g * 2)
        idx1 = one_group(g * 2 + 1)
        for dd in range(D):
            feat0 = pts_v[dd, pl.ds(g * 32, 16)]
            feat1 = pts_v[dd, pl.ds(g * 32 + 16, 16)]
            plsc.addupdate_scatter(hist_v, [idx0], feat0)
            plsc.addupdate_scatter(hist_v, [idx1], feat1)
            idx0 = idx0 + ones_iv
            idx1 = idx1 + ones_iv
        return carry

    lax.fori_loop(0, NG // 2, group, 0)

    for c in range(LHIST // 16):
        acc = hist_v[pl.ds(c * 16, 16)]
        for ln in range(1, 16):
            acc = acc + hist_v[pl.ds(ln * LHIST + c * 16, 16)]
        psum_v[pl.ds(c * 16, 16)] = acc
    cacc = cnt_v[pl.ds(0, 16)]
    for ln in range(1, 16):
        cacc = cacc + cnt_v[pl.ds(ln * 16, 16)]
    pcnt_v[...] = cacc.astype(jnp.int32)

    pltpu.sync_copy(psum_v, psums_hbm.at[wid])
    pltpu.sync_copy(pcnt_v, pcnts_hbm.at[wid])


def _sc_hist_call(batched_pts, voxel_sizes, bins):
    f = pl.kernel(
        _sc_hist_body,
        out_type=[jax.ShapeDtypeStruct((NW, LHIST), jnp.float32),
                  jax.ShapeDtypeStruct((NW, 16), jnp.int32)],
        mesh=plsc.VectorSubcoreMesh(core_axis_name="c", subcore_axis_name="s"),
        compiler_params=pltpu.CompilerParams(needs_layout_passes=False),
        scratch_types=[
            pltpu.VMEM((D, CHUNK), jnp.float32),
            pltpu.VMEM((CHUNK,), jnp.float32),
            pltpu.VMEM((256,), jnp.float32),
            pltpu.VMEM((16 * LHIST,), jnp.float32),
            pltpu.VMEM((16 * 16,), jnp.float32),
            pltpu.VMEM((LHIST,), jnp.float32),
            pltpu.VMEM((16,), jnp.int32),
        ],
    )
    return f(batched_pts, voxel_sizes, bins)


def _combine_body(ps_ref, pc_ref, s_ref, c_ref):
    s_ref[...] = jnp.sum(ps_ref[...], axis=0, keepdims=True)
    c_ref[...] = jnp.sum(pc_ref[...], axis=0, keepdims=True)


def _combine_call(psums, pcnts):
    return pl.pallas_call(
        _combine_body,
        out_shape=[jax.ShapeDtypeStruct((1, LHIST), jnp.float32),
                   jax.ShapeDtypeStruct((1, 16), jnp.int32)],
    )(psums, pcnts)


def kernel(batched_pts, Wv, Wk, Wq, Wo, bo):
    out, voxel3, bins128 = _attention_call(
        batched_pts, Wv, Wk, Wq, Wo, bo.reshape(1, D))
    voxel_sizes = voxel3.reshape(N, P)
    psums, pcnts = _sc_hist_call(batched_pts, voxel_sizes,
                                 bins128.reshape(256))
    s, c = _combine_call(psums, pcnts)
    bin_sums = s.reshape(BIN_SIZE, D)
    counts = c.reshape(16)[:BIN_SIZE]
    return out, voxel_sizes, counts, bin_sums


# speedup vs baseline: 2.0010x; 1.0006x over previous
import jax
import jax.numpy as jnp
import numpy as np
from jax import lax
from jax.experimental import pallas as pl
from jax.experimental.pallas import tpu as pltpu
from jax.experimental.pallas import tpu_sc as plsc

N, D, P, H = 8, 32, 4096, 1
HD = D // H
VOXEL_BASE = 0.05
VOXEL_RANGE = 0.1
BIN_SIZE = 10

BQ = 2048
NQB = P // BQ
_INV_SQRT_D = np.float32(0.176776692)


def _attn_body(pts_ref, wv_ref, wk_ref, wq_ref, wo_ref, bo_ref,
               out_ref, voxel_ref, bins_ref,
               qt_ref, kt_ref, vt_ref, cm_ref, gmm_ref):
    n = pl.program_id(0)
    qb = pl.program_id(1)
    pts = pts_ref[0]

    @pl.when(qb == 0)
    def _project():
        ptsb = pts.astype(jnp.bfloat16)
        qt_ref[...] = jax.lax.dot_general(
            wq_ref[...].astype(jnp.bfloat16), ptsb, (((1,), (0,)), ((), ())),
            preferred_element_type=jnp.float32).astype(jnp.bfloat16)
        kt_ref[...] = jax.lax.dot_general(
            wk_ref[...].astype(jnp.bfloat16), ptsb, (((1,), (0,)), ((), ())),
            preferred_element_type=jnp.float32).astype(jnp.bfloat16)
        vt_ref[...] = jax.lax.dot_general(
            wv_ref[...], pts, (((1,), (0,)), ((), ())),
            preferred_element_type=jnp.float32,
            precision=jax.lax.Precision.HIGHEST)

    q_blk = qt_ref[:, pl.ds(qb * BQ, BQ)]
    e = jax.lax.dot_general(
        q_blk, kt_ref[...], (((0,), (0,)), ((), ())),
        preferred_element_type=jnp.float32) * _INV_SQRT_D
    m = jnp.max(e, axis=1, keepdims=True)
    p = jnp.exp(e - m)
    l = jnp.sum(p, axis=1, keepdims=True)
    att = p * (1.0 / l)

    cm_blk = jnp.max(att, axis=0, keepdims=True)

    @pl.when(qb == 0)
    def _cm_init():
        cm_ref[...] = cm_blk

    @pl.when(qb > 0)
    def _cm_acc():
        cm_ref[...] = jnp.maximum(cm_ref[...], cm_blk)

    ov = jax.lax.dot_general(
        att, vt_ref[...], (((1,), (1,)), ((), ())),
        preferred_element_type=jnp.float32)
    out_ref[0] = jax.lax.dot_general(
        ov, wo_ref[...], (((1,), (1,)), ((), ())),
        preferred_element_type=jnp.float32,
        precision=jax.lax.Precision.HIGHEST) + bo_ref[...]

    @pl.when(qb == NQB - 1)
    def _voxel():
        cm = cm_ref[...]
        mn = jnp.min(cm)
        mx = jnp.max(cm)
        norm = (cm - mn) / (mx - mn)
        voxel = VOXEL_BASE + (1.0 - norm) * VOXEL_RANGE
        voxel_ref[0] = voxel
        vmn = jnp.min(voxel)
        vmx = jnp.max(voxel)

        @pl.when(n == 0)
        def _g_init():
            gmm_ref[0] = vmn
            gmm_ref[1] = vmx

        @pl.when(n > 0)
        def _g_acc():
            gmm_ref[0] = jnp.minimum(gmm_ref[0], vmn)
            gmm_ref[1] = jnp.maximum(gmm_ref[1], vmx)

        @pl.when(n == N - 1)
        def _bins():
            lane = lax.broadcasted_iota(jnp.int32, (1, 256), 1)
            i_f = (lane // 16).astype(jnp.float32)
            step = i_f / np.float32(BIN_SIZE)
            bins_ref[...] = gmm_ref[0] * (1.0 - step) + gmm_ref[1] * step


def _attention_call(batched_pts, Wv, Wk, Wq, Wo, bo2):
    return pl.pallas_call(
        _attn_body,
        grid=(N, NQB),
        in_specs=[
            pl.BlockSpec((1, D, P), lambda n, q: (n, 0, 0)),
            pl.BlockSpec((D, D), lambda n, q: (0, 0)),
            pl.BlockSpec((D, D), lambda n, q: (0, 0)),
            pl.BlockSpec((D, D), lambda n, q: (0, 0)),
            pl.BlockSpec((D, D), lambda n, q: (0, 0)),
            pl.BlockSpec((1, D), lambda n, q: (0, 0)),
        ],
        out_specs=[
            pl.BlockSpec((1, BQ, D), lambda n, q: (n, q, 0)),
            pl.BlockSpec((1, 1, P), lambda n, q: (n, 0, 0)),
            pl.BlockSpec((1, 256), lambda n, q: (0, 0)),
        ],
        out_shape=[
            jax.ShapeDtypeStruct((N, P, D), jnp.float32),
            jax.ShapeDtypeStruct((N, 1, P), jnp.float32),
            jax.ShapeDtypeStruct((1, 256), jnp.float32),
        ],
        scratch_shapes=[
            pltpu.VMEM((D, P), jnp.bfloat16),
            pltpu.VMEM((D, P), jnp.bfloat16),
            pltpu.VMEM((D, P), jnp.float32),
            pltpu.VMEM((1, P), jnp.float32),
            pltpu.SMEM((2,), jnp.float32),
        ],
    )(batched_pts, Wv, Wk, Wq, Wo, bo2)


NW = 32
CHUNK = P * N // NW
NG = CHUNK // 16
LHIST = BIN_SIZE * D


def _sc_hist_body(pts_hbm, vox_hbm, bins_hbm, psums_hbm, pcnts_hbm,
                  pts_v, vox_v, bins_v, hist_v, cnt_v, psum_v, pcnt_v):
    wid = lax.axis_index("c") * 16 + lax.axis_index("s")
    n = wid // 4
    off = (wid % 4) * CHUNK

    pltpu.sync_copy(bins_hbm, bins_v)
    pltpu.sync_copy(vox_hbm.at[n, pl.ds(off, CHUNK)], vox_v)
    pltpu.sync_copy(pts_hbm.at[n, :, pl.ds(off, CHUNK)], pts_v)

    zeros = jnp.zeros((16,), jnp.float32)
    zeros_i = jnp.zeros((16,), jnp.int32)
    ones_iv = jnp.full((16,), 1, jnp.int32)
    nine_iv = jnp.full((16,), BIN_SIZE - 1, jnp.int32)
    ten_iv = jnp.full((16,), BIN_SIZE, jnp.int32)
    dim_iv = jnp.full((16,), D, jnp.int32)
    iota = lax.iota(jnp.int32, 16)
    lane_hist = iota * jnp.full((16,), LHIST, jnp.int32)
    lane_cnt = iota * jnp.full((16,), 16, jnp.int32)

    ones_fv = jnp.full((16,), 1.0, jnp.float32)
    for j in range(16 * LHIST // 16):
        hist_v[pl.ds(j * 16, 16)] = zeros
    for j in range(16):
        cnt_v[pl.ds(j * 16, 16)] = zeros

    bcast_bins = [bins_v[pl.ds(i * 16, 16)] for i in range(BIN_SIZE + 1)]

    def one_group(g):
        v = vox_v[pl.ds(g * 16, 16)]
        c = lax.select(bcast_bins[0] <= v, ones_iv, zeros_i)
        for i in range(1, BIN_SIZE + 1):
            c = c + lax.select(bcast_bins[i] <= v, ones_iv, zeros_i)
        b = c - ones_iv
        bad = jnp.logical_or(b < zeros_i, b >= ten_iv)
        b = lax.select(bad, nine_iv, b)
        plsc.addupdate_scatter(cnt_v, [lane_cnt + b], ones_fv)
        return lane_hist + b * dim_iv

    def group(g, carry):
        idx0 = one_group(g * 2)
        idx1 = one_group(g * 2 + 1)
        for dd in range(D):
            feat0 = pts_v[dd, pl.ds(g * 32, 16)]
            feat1 = pts_v[dd, pl.ds(g * 32 + 16, 16)]
            plsc.addupdate_scatter(hist_v, [idx0], feat0)
            plsc.addupdate_scatter(hist_v, [idx1], feat1)
            idx0 = idx0 + ones_iv
            idx1 = idx1 + ones_iv
        return carry

    lax.fori_loop(0, NG // 2, group, 0)

    for c in range(LHIST // 16):
        acc = hist_v[pl.ds(c * 16, 16)]
        for ln in range(1, 16):
            acc = acc + hist_v[pl.ds(ln * LHIST + c * 16, 16)]
        psum_v[pl.ds(c * 16, 16)] = acc
    cacc = cnt_v[pl.ds(0, 16)]
    for ln in range(1, 16):
        cacc = cacc + cnt_v[pl.ds(ln * 16, 16)]
    pcnt_v[...] = cacc.astype(jnp.int32)

    pltpu.sync_copy(psum_v, psums_hbm.at[wid])
    pltpu.sync_copy(pcnt_v, pcnts_hbm.at[wid])


def _sc_hist_call(batched_pts, voxel_sizes, bins):
    f = pl.kernel(
        _sc_hist_body,
        out_type=[jax.ShapeDtypeStruct((NW, LHIST), jnp.float32),
                  jax.ShapeDtypeStruct((NW, 16), jnp.int32)],
        mesh=plsc.VectorSubcoreMesh(core_axis_name="c", subcore_axis_name="s"),
        compiler_params=pltpu.CompilerParams(needs_layout_passes=False),
        scratch_types=[
            pltpu.VMEM((D, CHUNK), jnp.float32),
            pltpu.VMEM((CHUNK,), jnp.float32),
            pltpu.VMEM((256,), jnp.float32),
            pltpu.VMEM((16 * LHIST,), jnp.float32),
            pltpu.VMEM((16 * 16,), jnp.float32),
            pltpu.VMEM((LHIST,), jnp.float32),
            pltpu.VMEM((16,), jnp.int32),
        ],
    )
    return f(batched_pts, voxel_sizes, bins)


def _combine_body(ps_ref, pc_ref, s_ref, c_ref):
    s_ref[...] = jnp.sum(ps_ref[...], axis=0, keepdims=True)
    c_ref[...] = jnp.sum(pc_ref[...], axis=0, keepdims=True)


def _combine_call(psums, pcnts):
    return pl.pallas_call(
        _combine_body,
        out_shape=[jax.ShapeDtypeStruct((1, LHIST), jnp.float32),
                   jax.ShapeDtypeStruct((1, 16), jnp.int32)],
    )(psums, pcnts)


def kernel(batched_pts, Wv, Wk, Wq, Wo, bo):
    out, voxel3, bins128 = _attention_call(
        batched_pts, Wv, Wk, Wq, Wo, bo.reshape(1, D))
    voxel_sizes = voxel3.reshape(N, P)
    psums, pcnts = _sc_hist_call(batched_pts, voxel_sizes,
                                 bins128.reshape(256))
    s, c = _combine_call(psums, pcnts)
    bin_sums = s.reshape(BIN_SIZE, D)
    counts = c.reshape(16)[:BIN_SIZE]
    return out, voxel_sizes, counts, bin_sums
